# baseline (speedup 1.0000x reference)
import jax
import jax.numpy as jnp
from jax import lax
from jax.experimental import pallas as pl
from jax.experimental.pallas import tpu as pltpu

_ROWS = 8
_NEG_INF = float("-inf")
_I32_MAX = jnp.iinfo(jnp.int32).max
_I32_MIN = jnp.iinfo(jnp.int32).min
_TIE_UNROLL = 16


def _floor_avg(a, b):
    return (a & b) + ((a ^ b) >> 1)


def _ceil_avg(a, b):
    return (a & b) + ((a ^ b) >> 1) + ((a ^ b) & 1)


def _to_key(b):
    return b ^ (lax.shift_right_arithmetic(b, 31) & jnp.int32(0x7FFFFFFF))


def _sampler_kernel(
    x_ref, t_ref, k_ref, p_ref, id_ref, o_ref, key_ref, prob_ref, hi16_ref, lo16_ref, cut_ref
):
    G, V = x_ref.shape
    _, VP = key_ref.shape
    tail = (V // 128) * 128

    key_ref[:, tail:] = jnp.full((G, VP - tail), _I32_MIN, jnp.int32)
    prob_ref[:, tail:] = jnp.zeros((G, VP - tail), jnp.float32)
    hi16_ref[:, tail:] = jnp.full((G, VP - tail), -(2 ** 15), jnp.int16)
    lo16_ref[:, tail:] = jnp.full((G, VP - tail), -(2 ** 15), jnp.int16)

    y = x_ref[...] / t_ref[...]
    o_ref[...] = y
    key = _to_key(lax.bitcast_convert_type(y, jnp.int32))
    key_ref[:, :V] = key
    hi16_ref[:, :V] = (key >> 16).astype(jnp.int16)
    lo16_ref[:, :V] = lax.bitcast_convert_type(
        ((key & 0xFFFF) ^ 0x8000).astype(jnp.uint16), jnp.int16
    )

    mx_key = jnp.max(key_ref[...], axis=1, keepdims=True)
    mx = lax.bitcast_convert_type(_to_key(mx_key), jnp.float32)

    idx = lax.broadcasted_iota(jnp.int32, (G, VP), 1)
    id_ref[...] = jnp.min(
        jnp.where(key_ref[...] == mx_key, idx, VP), axis=1, keepdims=True
    )

    k = jnp.clip(k_ref[...], 1, V)
    tp = p_ref[...]

    def not_done(lohi):
        lo, hi = lohi
        return jnp.any(lo < hi)

    def body1a(lohi):
        lo, hi = lohi
        mid = _ceil_avg(lo, hi)
        cnt = jnp.sum(
            (hi16_ref[...] >= mid.astype(jnp.int16)).astype(jnp.int32),
            axis=1,
            keepdims=True,
        )
        pred = cnt >= k
        return jnp.where(pred, mid, lo), jnp.where(pred, hi, mid - 1)

    h_lo0 = jnp.full((G, 1), -(2 ** 15) + 1, jnp.int32)
    h_hi0 = mx_key >> 16
    h_kv, _ = lax.while_loop(not_done, body1a, (h_lo0, h_hi0))

    h_kv16 = h_kv.astype(jnp.int16)
    in_cls = hi16_ref[...] == h_kv16
    cnt_gt = jnp.sum((hi16_ref[...] > h_kv16).astype(jnp.int32), axis=1, keepdims=True)
    k_in = k - cnt_gt

    def body1b(lohi):
        lo, hi = lohi
        mid = _ceil_avg(lo, hi)
        cnt = jnp.sum(
            (in_cls & (lo16_ref[...] >= mid.astype(jnp.int16))).astype(jnp.int32),
            axis=1,
            keepdims=True,
        )
        pred = cnt >= k_in
        return jnp.where(pred, mid, lo), jnp.where(pred, hi, mid - 1)

    l_lo0 = jnp.full((G, 1), -(2 ** 15), jnp.int32)
    l_hi0 = jnp.full((G, 1), 2 ** 15 - 1, jnp.int32)
    l_kv, _ = lax.while_loop(not_done, body1b, (l_lo0, l_hi0))

    kv = (h_kv << 16) | ((l_kv & 0xFFFF) ^ 0x8000)

    e = jnp.where(key_ref[:, :V] >= kv, jnp.exp(o_ref[...] - mx), 0.0)
    z = jnp.sum(e, axis=1, keepdims=True)
    prob_ref[:, :V] = e / z

    def body2(lohi):
        lo, hi = lohi
        mid = _floor_avg(lo, hi)
        g = jnp.sum(jnp.where(key_ref[...] > mid, prob_ref[...], 0.0), axis=1, keepdims=True)
        pred = g <= tp
        return jnp.where(pred, lo, mid + 1), jnp.where(pred, mid, hi)

    ws, _ = lax.while_loop(not_done, body2, (kv, mx_key))

    thr = jnp.maximum(kv, ws)
    keep_pad = key_ref[...] >= thr

    w_cls = jnp.min(jnp.where(keep_pad, key_ref[...], _I32_MAX), axis=1, keepdims=True)
    is_w = key_ref[...] == w_cls
    t_w = jnp.sum(is_w.astype(jnp.int32), axis=1, keepdims=True)
    g_w = jnp.sum(jnp.where(key_ref[...] > w_cls, prob_ref[...], 0.0), axis=1, keepdims=True)
    p_w = jnp.max(jnp.where(is_w, prob_ref[...], 0.0), axis=1, keepdims=True)

    c = jnp.ones_like(t_w)
    s = g_w
    for q in range(2, _TIE_UNROLL + 1):
        s = s + p_w
        c = c + ((s <= tp) & (q <= t_w)).astype(jnp.int32)

    cut_ref[...] = jnp.full((G, 1), VP, jnp.int32)

    @pl.when(jnp.any((t_w > 1) & (c < t_w)))
    def _tie_cut():
        last = jnp.full((G, 1), -1, jnp.int32)
        cut = jnp.full((G, 1), VP, jnp.int32)
        for q in range(1, _TIE_UNROLL + 1):
            nxt = jnp.min(jnp.where(is_w & (idx > last), idx, VP), axis=1, keepdims=True)
            cut = jnp.where(c == q, nxt, cut)
            last = nxt
        cut_ref[...] = cut

    keep = keep_pad[:, :V] & ~(is_w[:, :V] & (idx[:, :V] > cut_ref[...]))
    o_ref[...] = jnp.where(keep, o_ref[...], _NEG_INF)


def kernel(logits, temperature, top_k, top_p):
    B, V = logits.shape
    VP = ((V + 127) // 128) * 128
    logits = logits.astype(jnp.float32)
    grid = (B // _ROWS,)
    row_spec = pl.BlockSpec((_ROWS, 1), lambda i: (i, 0))
    ids, out = pl.pallas_call(
        _sampler_kernel,
        grid=grid,
        in_specs=[
            pl.BlockSpec((_ROWS, V), lambda i: (i, 0)),
            row_spec,
            row_spec,
            row_spec,
        ],
        out_specs=[row_spec, pl.BlockSpec((_ROWS, V), lambda i: (i, 0))],
        out_shape=[
            jax.ShapeDtypeStruct((B, 1), jnp.int32),
            jax.ShapeDtypeStruct((B, V), jnp.float32),
        ],
        scratch_shapes=[
            pltpu.VMEM((_ROWS, VP), jnp.int32),
            pltpu.VMEM((_ROWS, VP), jnp.float32),
            pltpu.VMEM((_ROWS, VP), jnp.int16),
            pltpu.VMEM((_ROWS, VP), jnp.int16),
            pltpu.VMEM((_ROWS, 1), jnp.int32),
        ],
        compiler_params=pltpu.CompilerParams(
            dimension_semantics=("parallel",),
        ),
    )(
        logits,
        temperature.astype(jnp.float32)[:, None],
        top_k.astype(jnp.int32)[:, None],
        top_p.astype(jnp.float32)[:, None],
    )
    return ids, out


# speedup vs baseline: 1.5208x; 1.5208x over previous
import jax
import jax.numpy as jnp
from jax import lax
from jax.experimental import pallas as pl
from jax.experimental.pallas import tpu as pltpu

_ROWS = 16
_NEG_INF = float("-inf")
_I32_MAX = jnp.iinfo(jnp.int32).max
_I32_MIN = jnp.iinfo(jnp.int32).min
_TIE_UNROLL = 16


def _floor_avg(a, b):
    return (a & b) + ((a ^ b) >> 1)


def _ceil_avg(a, b):
    return (a & b) + ((a ^ b) >> 1) + ((a ^ b) & 1)


def _to_key(b):
    return b ^ (lax.shift_right_arithmetic(b, 31) & jnp.int32(0x7FFFFFFF))


def _sampler_kernel(x_ref, t_ref, k_ref, p_ref, id_ref, o_ref, key_ref, prob_ref, cut_ref):
    G, V = x_ref.shape
    _, VP = key_ref.shape
    tail = (V // 128) * 128

    key_ref[:, tail:] = jnp.full((G, VP - tail), _I32_MIN, jnp.int32)
    prob_ref[:, tail:] = jnp.zeros((G, VP - tail), jnp.float32)

    y = x_ref[...] / t_ref[...]
    o_ref[...] = y
    key_ref[:, :V] = _to_key(lax.bitcast_convert_type(y, jnp.int32))

    mx_key = jnp.max(key_ref[...], axis=1, keepdims=True)
    mx = lax.bitcast_convert_type(_to_key(mx_key), jnp.float32)

    idx = lax.broadcasted_iota(jnp.int32, (G, VP), 1)
    id_ref[...] = jnp.min(
        jnp.where(key_ref[...] == mx_key, idx, VP), axis=1, keepdims=True
    )

    k = jnp.clip(k_ref[...], 1, V)
    tp = p_ref[...]

    def not_done(lohi):
        lo, hi = lohi
        return jnp.any(lo < hi)

    def body1(lohi):
        lo, hi = lohi
        mid = _ceil_avg(lo, hi)
        cnt = jnp.sum((key_ref[...] >= mid).astype(jnp.int32), axis=1, keepdims=True)
        pred = cnt >= k
        return jnp.where(pred, mid, lo), jnp.where(pred, hi, mid - 1)

    lo0 = jnp.full((G, 1), _I32_MIN + 1, jnp.int32)
    kv, _ = lax.while_loop(not_done, body1, (lo0, mx_key))

    e = jnp.where(key_ref[:, :V] >= kv, jnp.exp(o_ref[...] - mx), 0.0)
    z = jnp.sum(e, axis=1, keepdims=True)
    prob_ref[:, :V] = e / z

    def body2(lohi):
        lo, hi = lohi
        mid = _floor_avg(lo, hi)
        g = jnp.sum(jnp.where(key_ref[...] > mid, prob_ref[...], 0.0), axis=1, keepdims=True)
        pred = g <= tp
        return jnp.where(pred, lo, mid + 1), jnp.where(pred, mid, hi)

    ws, _ = lax.while_loop(not_done, body2, (kv, mx_key))

    thr = jnp.maximum(kv, ws)
    keep_pad = key_ref[...] >= thr

    w_cls = jnp.min(jnp.where(keep_pad, key_ref[...], _I32_MAX), axis=1, keepdims=True)
    is_w = key_ref[...] == w_cls
    t_w = jnp.sum(is_w.astype(jnp.int32), axis=1, keepdims=True)
    g_w = jnp.sum(jnp.where(key_ref[...] > w_cls, prob_ref[...], 0.0), axis=1, keepdims=True)
    p_w = jnp.max(jnp.where(is_w, prob_ref[...], 0.0), axis=1, keepdims=True)

    c = jnp.ones_like(t_w)
    s = g_w
    for q in range(2, _TIE_UNROLL + 1):
        s = s + p_w
        c = c + ((s <= tp) & (q <= t_w)).astype(jnp.int32)

    cut_ref[...] = jnp.full((G, 1), VP, jnp.int32)

    @pl.when(jnp.any((t_w > 1) & (c < t_w)))
    def _tie_cut():
        last = jnp.full((G, 1), -1, jnp.int32)
        cut = jnp.full((G, 1), VP, jnp.int32)
        for q in range(1, _TIE_UNROLL + 1):
            nxt = jnp.min(jnp.where(is_w & (idx > last), idx, VP), axis=1, keepdims=True)
            cut = jnp.where(c == q, nxt, cut)
            last = nxt
        cut_ref[...] = cut

    keep = keep_pad[:, :V] & ~(is_w[:, :V] & (idx[:, :V] > cut_ref[...]))
    o_ref[...] = jnp.where(keep, o_ref[...], _NEG_INF)


def kernel(logits, temperature, top_k, top_p):
    B, V = logits.shape
    VP = ((V + 127) // 128) * 128
    logits = logits.astype(jnp.float32)
    grid = (B // _ROWS,)
    row_spec = pl.BlockSpec((_ROWS, 1), lambda i: (i, 0))
    ids, out = pl.pallas_call(
        _sampler_kernel,
        grid=grid,
        in_specs=[
            pl.BlockSpec((_ROWS, V), lambda i: (i, 0)),
            row_spec,
            row_spec,
            row_spec,
        ],
        out_specs=[row_spec, pl.BlockSpec((_ROWS, V), lambda i: (i, 0))],
        out_shape=[
            jax.ShapeDtypeStruct((B, 1), jnp.int32),
            jax.ShapeDtypeStruct((B, V), jnp.float32),
        ],
        scratch_shapes=[
            pltpu.VMEM((_ROWS, VP), jnp.int32),
            pltpu.VMEM((_ROWS, VP), jnp.float32),
            pltpu.VMEM((_ROWS, 1), jnp.int32),
        ],
        compiler_params=pltpu.CompilerParams(
            dimension_semantics=("parallel",),
        ),
    )(
        logits,
        temperature.astype(jnp.float32)[:, None],
        top_k.astype(jnp.int32)[:, None],
        top_p.astype(jnp.float32)[:, None],
    )
    return ids, out
